# baseline (speedup 1.0000x reference)
import jax
import jax.numpy as jnp
from jax import lax
from jax.experimental import pallas as pl
from jax.experimental.pallas import tpu as pltpu

_ALPHA = 0.25


def _tc_body(logits_ref, boxes_ref, tboxes_ref, sidx_ref, tlab_ref,
             ce_ref, l1_ref, gi_ref):
    b = pl.program_id(0)

    @pl.when(b == 0)
    def _init():
        ce_ref[...] = jnp.zeros((1, 1), jnp.float32)
        l1_ref[...] = jnp.zeros((1, 1), jnp.float32)
        gi_ref[...] = jnp.zeros((1, 1), jnp.float32)

    x = logits_ref[0]
    sidx = sidx_ref[0]
    tlab = tlab_ref[0]
    Q, C = x.shape
    T = sidx.shape[1]

    sidx_col = sidx.reshape(T, 1)
    tlab_col = tlab.reshape(T, 1)
    tio = lax.broadcasted_iota(jnp.int32, (T, Q), 0)
    qio = lax.broadcasted_iota(jnp.int32, (T, Q), 1)
    eq = sidx_col == qio
    t_last = jnp.max(jnp.where(eq, tio, -1), axis=0, keepdims=True)
    pick = eq & (tio == t_last)
    lab_q = jnp.sum(jnp.where(pick, tlab_col, 0), axis=0, keepdims=True)
    cls_q = jnp.where(t_last >= 0, lab_q, C)

    cio = lax.broadcasted_iota(jnp.int32, (Q, C), 1)
    m = (cio == cls_q.reshape(Q, 1)).astype(jnp.float32)
    p = 1.0 / (1.0 + jnp.exp(-x))
    ce = jnp.maximum(x, 0.0) - x * m + jnp.log1p(jnp.exp(-jnp.abs(x)))
    p_t = p * m + (1.0 - p) * (1.0 - m)
    w = _ALPHA * m + (1.0 - _ALPHA) * (1.0 - m)
    om = 1.0 - p_t
    ce_ref[...] += jnp.sum(w * ce * om * om).reshape(1, 1)

    pb = boxes_ref[0]
    tb = tboxes_ref[0]
    sb = jnp.dot(eq.astype(jnp.float32), pb,
                 preferred_element_type=jnp.float32)
    l1_ref[...] += jnp.sum(jnp.abs(sb - tb)).reshape(1, 1)

    def xyxy(bx):
        cx, cy, w_, h_ = bx[:, 0], bx[:, 1], bx[:, 2], bx[:, 3]
        return cx - 0.5 * w_, cy - 0.5 * h_, cx + 0.5 * w_, cy + 0.5 * h_

    sx0, sy0, sx1, sy1 = xyxy(sb)
    tx0, ty0, tx1, ty1 = xyxy(tb)
    a1 = (sx1 - sx0) * (sy1 - sy0)
    a2 = (tx1 - tx0) * (ty1 - ty0)
    iw = jnp.maximum(jnp.minimum(sx1, tx1) - jnp.maximum(sx0, tx0), 0.0)
    ih = jnp.maximum(jnp.minimum(sy1, ty1) - jnp.maximum(sy0, ty0), 0.0)
    inter = iw * ih
    union = a1 + a2 - inter
    iou = inter / union
    ew = jnp.maximum(sx1, tx1) - jnp.minimum(sx0, tx0)
    eh = jnp.maximum(sy1, ty1) - jnp.minimum(sy0, ty0)
    ae = ew * eh
    giou = iou - (ae - union) / ae
    gi_ref[...] += jnp.sum(1.0 - giou).reshape(1, 1)


def kernel(pred_logits, pred_boxes, tgt_boxes, src_idx, tgt_labels):
    B, Q, C = pred_logits.shape
    T = src_idx.shape[1]
    f32 = jnp.float32
    ce, l1, gi = pl.pallas_call(
        _tc_body,
        grid=(B,),
        in_specs=[
            pl.BlockSpec((1, Q, C), lambda b: (b, 0, 0)),
            pl.BlockSpec((1, Q, 4), lambda b: (b, 0, 0)),
            pl.BlockSpec((1, T, 4), lambda b: (b, 0, 0)),
            pl.BlockSpec((1, 1, T), lambda b: (b, 0, 0)),
            pl.BlockSpec((1, 1, T), lambda b: (b, 0, 0)),
        ],
        out_specs=[pl.BlockSpec((1, 1), lambda b: (0, 0))] * 3,
        out_shape=[jax.ShapeDtypeStruct((1, 1), f32)] * 3,
    )(pred_logits, pred_boxes, tgt_boxes,
      src_idx.reshape(B, 1, T), tgt_labels.reshape(B, 1, T))
    nb = float(B * T)
    return (ce[0, 0] / nb, l1[0, 0] / nb, gi[0, 0] / nb)


# speedup vs baseline: 1.6620x; 1.6620x over previous
import jax
import jax.numpy as jnp
from jax import lax
from jax.experimental import pallas as pl
from jax.experimental.pallas import tpu as pltpu

_ALPHA = 0.25
_BB = 4


def _tc_body(logits_ref, boxes_ref, tboxes_ref, sidx_ref, tlab_ref,
             ce_ref, l1_ref, gi_ref):
    g = pl.program_id(0)

    @pl.when(g == 0)
    def _init():
        ce_ref[...] = jnp.zeros((1, 1), jnp.float32)
        l1_ref[...] = jnp.zeros((1, 1), jnp.float32)
        gi_ref[...] = jnp.zeros((1, 1), jnp.float32)

    x = logits_ref[...]
    sidx = sidx_ref[...]
    tlab = tlab_ref[...]
    BB, Q, C = x.shape
    T = sidx.shape[2]

    sidx_c = sidx.reshape(BB, T, 1)
    tlab_c = tlab.reshape(BB, T, 1)
    tio = lax.broadcasted_iota(jnp.int32, (BB, T, Q), 1)
    qio = lax.broadcasted_iota(jnp.int32, (BB, T, Q), 2)
    eq = sidx_c == qio
    t_last = jnp.max(jnp.where(eq, tio, -1), axis=1)
    pick = eq & (tio == t_last[:, None, :])
    lab_q = jnp.sum(jnp.where(pick, tlab_c, 0), axis=1)
    cls_q = jnp.where(t_last >= 0, lab_q, C)

    cio = lax.broadcasted_iota(jnp.int32, (BB, Q, C), 2)
    mb = cio == cls_q[:, :, None]
    mf = mb.astype(jnp.float32)
    e = jnp.exp(-jnp.abs(x))
    l = jnp.log1p(e)
    mx = jnp.maximum(x, 0.0)
    ce = mx - x * mf + l
    r = 1.0 / (1.0 + e)
    pos = x >= 0.0
    p = jnp.where(pos, r, 1.0 - r)
    om = jnp.where(mb, 1.0 - p, p)
    w = jnp.where(mb, _ALPHA, 1.0 - _ALPHA)
    ce_ref[...] += jnp.sum(w * ce * om * om).reshape(1, 1)

    l1_acc = 0.0
    gi_acc = 0.0
    for i in range(BB):
        pb = boxes_ref[i]
        tb = tboxes_ref[i]
        sb = jnp.dot(eq[i].astype(jnp.float32), pb,
                     preferred_element_type=jnp.float32)
        l1_acc += jnp.sum(jnp.abs(sb - tb))

        sx0, sy0 = sb[:, 0] - 0.5 * sb[:, 2], sb[:, 1] - 0.5 * sb[:, 3]
        sx1, sy1 = sb[:, 0] + 0.5 * sb[:, 2], sb[:, 1] + 0.5 * sb[:, 3]
        tx0, ty0 = tb[:, 0] - 0.5 * tb[:, 2], tb[:, 1] - 0.5 * tb[:, 3]
        tx1, ty1 = tb[:, 0] + 0.5 * tb[:, 2], tb[:, 1] + 0.5 * tb[:, 3]
        a1 = (sx1 - sx0) * (sy1 - sy0)
        a2 = (tx1 - tx0) * (ty1 - ty0)
        iw = jnp.maximum(jnp.minimum(sx1, tx1) - jnp.maximum(sx0, tx0), 0.0)
        ih = jnp.maximum(jnp.minimum(sy1, ty1) - jnp.maximum(sy0, ty0), 0.0)
        inter = iw * ih
        union = a1 + a2 - inter
        iou = inter / union
        ew = jnp.maximum(sx1, tx1) - jnp.minimum(sx0, tx0)
        eh = jnp.maximum(sy1, ty1) - jnp.minimum(sy0, ty0)
        ae = ew * eh
        giou = iou - (ae - union) / ae
        gi_acc += jnp.sum(1.0 - giou)
    l1_ref[...] += l1_acc.reshape(1, 1)
    gi_ref[...] += gi_acc.reshape(1, 1)


def kernel(pred_logits, pred_boxes, tgt_boxes, src_idx, tgt_labels):
    B, Q, C = pred_logits.shape
    T = src_idx.shape[1]
    f32 = jnp.float32
    ce, l1, gi = pl.pallas_call(
        _tc_body,
        grid=(B // _BB,),
        in_specs=[
            pl.BlockSpec((_BB, Q, C), lambda g: (g, 0, 0)),
            pl.BlockSpec((_BB, Q, 4), lambda g: (g, 0, 0)),
            pl.BlockSpec((_BB, T, 4), lambda g: (g, 0, 0)),
            pl.BlockSpec((_BB, 1, T), lambda g: (g, 0, 0)),
            pl.BlockSpec((_BB, 1, T), lambda g: (g, 0, 0)),
        ],
        out_specs=[pl.BlockSpec((1, 1), lambda g: (0, 0))] * 3,
        out_shape=[jax.ShapeDtypeStruct((1, 1), f32)] * 3,
    )(pred_logits, pred_boxes, tgt_boxes,
      src_idx.reshape(B, 1, T), tgt_labels.reshape(B, 1, T))
    nb = float(B * T)
    return (ce[0, 0] / nb, l1[0, 0] / nb, gi[0, 0] / nb)


# speedup vs baseline: 1.6820x; 1.0120x over previous
import jax
import jax.numpy as jnp
from jax import lax
from jax.experimental import pallas as pl
from jax.experimental.pallas import tpu as pltpu

_ALPHA = 0.25
_BB = 4


def _tc_body(logits_ref, boxes_ref, tboxes_ref, sidx_ref, tlab_ref,
             ce_ref, l1_ref, gi_ref):
    g = pl.program_id(0)

    @pl.when(g == 0)
    def _init():
        ce_ref[...] = jnp.zeros((1, 1), jnp.float32)
        l1_ref[...] = jnp.zeros((1, 1), jnp.float32)
        gi_ref[...] = jnp.zeros((1, 1), jnp.float32)

    x = logits_ref[...]
    sidx = sidx_ref[...]
    tlab = tlab_ref[...]
    BB, Q, C = x.shape
    T = sidx.shape[2]

    sidx_c = sidx.reshape(BB, T, 1)
    tlab_c = tlab.reshape(BB, T, 1)
    tio = lax.broadcasted_iota(jnp.int32, (BB, T, Q), 1)
    qio = lax.broadcasted_iota(jnp.int32, (BB, T, Q), 2)
    eq = sidx_c == qio
    t_last = jnp.max(jnp.where(eq, tio, -1), axis=1)
    pick = eq & (tio == t_last[:, None, :])
    lab_q = jnp.sum(jnp.where(pick, tlab_c, 0), axis=1)
    cls_q = jnp.where(t_last >= 0, lab_q, C)

    cio = lax.broadcasted_iota(jnp.int32, (BB, Q, C), 2)
    mb = cio == cls_q[:, :, None]
    mf = mb.astype(jnp.float32)
    e = jnp.exp(-jnp.abs(x))
    d = 1.0 + e
    l = jnp.log(d)
    mx = jnp.maximum(x, 0.0)
    ce = mx - x * mf + l
    r = 1.0 / d
    pos = x >= 0.0
    p = jnp.where(pos, r, 1.0 - r)
    om = jnp.where(mb, 1.0 - p, p)
    w = jnp.where(mb, _ALPHA, 1.0 - _ALPHA)
    ce_ref[...] += jnp.sum(w * ce * om * om).reshape(1, 1)

    l1_acc = 0.0
    gi_acc = 0.0
    for i in range(BB):
        pb = boxes_ref[i]
        tb = tboxes_ref[i]
        sb = jnp.dot(eq[i].astype(jnp.float32), pb,
                     preferred_element_type=jnp.float32)
        l1_acc += jnp.sum(jnp.abs(sb - tb))

        sx0, sy0 = sb[:, 0] - 0.5 * sb[:, 2], sb[:, 1] - 0.5 * sb[:, 3]
        sx1, sy1 = sb[:, 0] + 0.5 * sb[:, 2], sb[:, 1] + 0.5 * sb[:, 3]
        tx0, ty0 = tb[:, 0] - 0.5 * tb[:, 2], tb[:, 1] - 0.5 * tb[:, 3]
        tx1, ty1 = tb[:, 0] + 0.5 * tb[:, 2], tb[:, 1] + 0.5 * tb[:, 3]
        a1 = (sx1 - sx0) * (sy1 - sy0)
        a2 = (tx1 - tx0) * (ty1 - ty0)
        iw = jnp.maximum(jnp.minimum(sx1, tx1) - jnp.maximum(sx0, tx0), 0.0)
        ih = jnp.maximum(jnp.minimum(sy1, ty1) - jnp.maximum(sy0, ty0), 0.0)
        inter = iw * ih
        union = a1 + a2 - inter
        iou = inter / union
        ew = jnp.maximum(sx1, tx1) - jnp.minimum(sx0, tx0)
        eh = jnp.maximum(sy1, ty1) - jnp.minimum(sy0, ty0)
        ae = ew * eh
        giou = iou - (ae - union) / ae
        gi_acc += jnp.sum(1.0 - giou)
    l1_ref[...] += l1_acc.reshape(1, 1)
    gi_ref[...] += gi_acc.reshape(1, 1)


def kernel(pred_logits, pred_boxes, tgt_boxes, src_idx, tgt_labels):
    B, Q, C = pred_logits.shape
    T = src_idx.shape[1]
    f32 = jnp.float32
    ce, l1, gi = pl.pallas_call(
        _tc_body,
        grid=(B // _BB,),
        in_specs=[
            pl.BlockSpec((_BB, Q, C), lambda g: (g, 0, 0)),
            pl.BlockSpec((_BB, Q, 4), lambda g: (g, 0, 0)),
            pl.BlockSpec((_BB, T, 4), lambda g: (g, 0, 0)),
            pl.BlockSpec((_BB, 1, T), lambda g: (g, 0, 0)),
            pl.BlockSpec((_BB, 1, T), lambda g: (g, 0, 0)),
        ],
        out_specs=[pl.BlockSpec((1, 1), lambda g: (0, 0))] * 3,
        out_shape=[jax.ShapeDtypeStruct((1, 1), f32)] * 3,
    )(pred_logits, pred_boxes, tgt_boxes,
      src_idx.reshape(B, 1, T), tgt_labels.reshape(B, 1, T))
    nb = float(B * T)
    return (ce[0, 0] / nb, l1[0, 0] / nb, gi[0, 0] / nb)


# speedup vs baseline: 1.9104x; 1.1358x over previous
import jax
import jax.numpy as jnp
from jax import lax
from jax.experimental import pallas as pl
from jax.experimental.pallas import tpu as pltpu

_ALPHA = 0.25
_BB = 8


def _tc_body(logits_ref, boxes_ref, tboxes_ref, sidx_ref, tlab_ref,
             ce_ref, l1_ref, gi_ref, *, inv_nb):
    g = pl.program_id(0)

    @pl.when(g == 0)
    def _init():
        ce_ref[...] = jnp.zeros((1, 1), jnp.float32)
        l1_ref[...] = jnp.zeros((1, 1), jnp.float32)
        gi_ref[...] = jnp.zeros((1, 1), jnp.float32)

    x = logits_ref[...]
    sidx = sidx_ref[...]
    tlab = tlab_ref[...]
    BB, Q, C = x.shape
    T = sidx.shape[1]

    sidx_c = sidx.reshape(BB, T, 1)
    tlab_c = tlab.reshape(BB, T, 1)
    tio = lax.broadcasted_iota(jnp.int32, (BB, T, Q), 1)
    qio = lax.broadcasted_iota(jnp.int32, (BB, T, Q), 2)
    eq = sidx_c == qio
    t_last = jnp.max(jnp.where(eq, tio, -1), axis=1)
    pick = eq & (tio == t_last[:, None, :])
    lab_q = jnp.sum(jnp.where(pick, tlab_c, 0), axis=1)
    cls_q = jnp.where(t_last >= 0, lab_q, C)

    cio = lax.broadcasted_iota(jnp.int32, (BB, Q, C), 2)
    mb = cio == cls_q[:, :, None]
    mf = mb.astype(jnp.float32)
    e = jnp.exp(-jnp.abs(x))
    d = 1.0 + e
    l = jnp.log(d)
    mx = jnp.maximum(x, 0.0)
    ce = mx - x * mf + l
    r = 1.0 / d
    pos = x >= 0.0
    p = jnp.where(pos, r, 1.0 - r)
    om = jnp.where(mb, 1.0 - p, p)
    w = jnp.where(mb, _ALPHA, 1.0 - _ALPHA)
    ce_ref[...] += (jnp.sum(w * ce * om * om) * inv_nb).reshape(1, 1)

    l1_acc = 0.0
    gi_acc = 0.0
    for i in range(BB):
        pb = boxes_ref[i]
        tb = tboxes_ref[i]
        sb = jnp.dot(eq[i].astype(jnp.float32), pb,
                     preferred_element_type=jnp.float32)
        l1_acc += jnp.sum(jnp.abs(sb - tb))

        sx0, sy0 = sb[:, 0] - 0.5 * sb[:, 2], sb[:, 1] - 0.5 * sb[:, 3]
        sx1, sy1 = sb[:, 0] + 0.5 * sb[:, 2], sb[:, 1] + 0.5 * sb[:, 3]
        tx0, ty0 = tb[:, 0] - 0.5 * tb[:, 2], tb[:, 1] - 0.5 * tb[:, 3]
        tx1, ty1 = tb[:, 0] + 0.5 * tb[:, 2], tb[:, 1] + 0.5 * tb[:, 3]
        a1 = (sx1 - sx0) * (sy1 - sy0)
        a2 = (tx1 - tx0) * (ty1 - ty0)
        iw = jnp.maximum(jnp.minimum(sx1, tx1) - jnp.maximum(sx0, tx0), 0.0)
        ih = jnp.maximum(jnp.minimum(sy1, ty1) - jnp.maximum(sy0, ty0), 0.0)
        inter = iw * ih
        union = a1 + a2 - inter
        iou = inter / union
        ew = jnp.maximum(sx1, tx1) - jnp.minimum(sx0, tx0)
        eh = jnp.maximum(sy1, ty1) - jnp.minimum(sy0, ty0)
        ae = ew * eh
        giou = iou - (ae - union) / ae
        gi_acc += jnp.sum(1.0 - giou)
    l1_ref[...] += (l1_acc * inv_nb).reshape(1, 1)
    gi_ref[...] += (gi_acc * inv_nb).reshape(1, 1)


def kernel(pred_logits, pred_boxes, tgt_boxes, src_idx, tgt_labels):
    B, Q, C = pred_logits.shape
    T = src_idx.shape[1]
    f32 = jnp.float32
    import functools
    body = functools.partial(_tc_body, inv_nb=1.0 / float(B * T))
    ce, l1, gi = pl.pallas_call(
        body,
        grid=(B // _BB,),
        in_specs=[
            pl.BlockSpec((_BB, Q, C), lambda g: (g, 0, 0)),
            pl.BlockSpec((_BB, Q, 4), lambda g: (g, 0, 0)),
            pl.BlockSpec((_BB, T, 4), lambda g: (g, 0, 0)),
            pl.BlockSpec((_BB, T), lambda g: (g, 0)),
            pl.BlockSpec((_BB, T), lambda g: (g, 0)),
        ],
        out_specs=[pl.BlockSpec((1, 1), lambda g: (0, 0))] * 3,
        out_shape=[jax.ShapeDtypeStruct((1, 1), f32)] * 3,
    )(pred_logits, pred_boxes, tgt_boxes, src_idx, tgt_labels)
    return (ce[0, 0], l1[0, 0], gi[0, 0])


# speedup vs baseline: 4.0758x; 2.1335x over previous
import functools

import jax
import jax.numpy as jnp
from jax import lax
from jax.experimental import pallas as pl
from jax.experimental.pallas import tpu as pltpu

_ALPHA = 0.25
_BB = 8


def _dense_body(xt_ref, sit_ref, tlt_ref, ce_ref, cls_ref, *, inv_nb):
    g = pl.program_id(0)
    C, BB, Q = xt_ref.shape
    T, B = sit_ref.shape

    @pl.when(g == 0)
    def _first():
        ce_ref[...] = jnp.zeros((1, 1), jnp.float32)
        sit = sit_ref[...]
        tlt = tlt_ref[...]
        qio = lax.broadcasted_iota(jnp.int32, (T, B, Q), 2)
        tio = lax.broadcasted_iota(jnp.int32, (T, B, Q), 0)
        eq = sit[:, :, None] == qio
        t_last = jnp.max(jnp.where(eq, tio, -1), axis=0)
        pick = eq & (tio == t_last[None])
        lab = jnp.sum(jnp.where(pick, tlt[:, :, None], 0), axis=0)
        cls_ref[...] = jnp.where(t_last >= 0, lab, C)

    x = xt_ref[...]
    cls_blk = cls_ref[pl.ds(pl.multiple_of(g * BB, BB), BB), :]
    cio = lax.broadcasted_iota(jnp.int32, (C, BB, Q), 0)
    mb = cio == cls_blk[None]
    mf = mb.astype(jnp.float32)
    e = jnp.exp(-jnp.abs(x))
    d = 1.0 + e
    l = jnp.log(d)
    ce = jnp.maximum(x, 0.0) - x * mf + l
    r = 1.0 / d
    p = jnp.where(x >= 0.0, r, 1.0 - r)
    om = jnp.where(mb, 1.0 - p, p)
    w = jnp.where(mb, _ALPHA, 1.0 - _ALPHA)
    ce_ref[...] += (jnp.sum(w * ce * om * om) * inv_nb).reshape(1, 1)


def _box_body(pbt_ref, tbt_ref, sit_ref, l1_ref, gi_ref, *, inv_nb):
    B, _, Q = pbt_ref.shape
    T = sit_ref.shape[0]
    sit = sit_ref[...]
    qio = lax.broadcasted_iota(jnp.int32, (T, B, Q), 2)
    sel = (sit[:, :, None] == qio).astype(jnp.float32)

    def coord(k):
        src = jnp.sum(sel * pbt_ref[:, k, :][None], axis=2)
        return src, tbt_ref[:, k, :]

    scx, tcx = coord(0)
    scy, tcy = coord(1)
    sw, tw = coord(2)
    sh, th = coord(3)
    l1 = (jnp.abs(scx - tcx) + jnp.abs(scy - tcy)
          + jnp.abs(sw - tw) + jnp.abs(sh - th))
    l1_ref[...] = (jnp.sum(l1) * inv_nb).reshape(1, 1)

    sx0, sx1 = scx - 0.5 * sw, scx + 0.5 * sw
    sy0, sy1 = scy - 0.5 * sh, scy + 0.5 * sh
    tx0, tx1 = tcx - 0.5 * tw, tcx + 0.5 * tw
    ty0, ty1 = tcy - 0.5 * th, tcy + 0.5 * th
    a1 = (sx1 - sx0) * (sy1 - sy0)
    a2 = (tx1 - tx0) * (ty1 - ty0)
    iw = jnp.maximum(jnp.minimum(sx1, tx1) - jnp.maximum(sx0, tx0), 0.0)
    ih = jnp.maximum(jnp.minimum(sy1, ty1) - jnp.maximum(sy0, ty0), 0.0)
    inter = iw * ih
    union = a1 + a2 - inter
    iou = inter / union
    ew = jnp.maximum(sx1, tx1) - jnp.minimum(sx0, tx0)
    eh = jnp.maximum(sy1, ty1) - jnp.minimum(sy0, ty0)
    ae = ew * eh
    giou = iou - (ae - union) / ae
    gi_ref[...] = (jnp.sum(1.0 - giou) * inv_nb).reshape(1, 1)


def kernel(pred_logits, pred_boxes, tgt_boxes, src_idx, tgt_labels):
    B, Q, C = pred_logits.shape
    T = src_idx.shape[1]
    f32 = jnp.float32
    inv_nb = 1.0 / float(B * T)

    xt = jnp.transpose(pred_logits, (2, 0, 1))
    sit = jnp.transpose(src_idx, (1, 0))
    tlt = jnp.transpose(tgt_labels, (1, 0))
    pbt = jnp.transpose(pred_boxes, (0, 2, 1))
    tbt = jnp.transpose(tgt_boxes, (1, 2, 0))

    ce = pl.pallas_call(
        functools.partial(_dense_body, inv_nb=inv_nb),
        grid=(B // _BB,),
        in_specs=[
            pl.BlockSpec((C, _BB, Q), lambda g: (0, g, 0)),
            pl.BlockSpec((T, B), lambda g: (0, 0)),
            pl.BlockSpec((T, B), lambda g: (0, 0)),
        ],
        out_specs=pl.BlockSpec((1, 1), lambda g: (0, 0)),
        out_shape=jax.ShapeDtypeStruct((1, 1), f32),
        scratch_shapes=[pltpu.VMEM((B, Q), jnp.int32)],
    )(xt, sit, tlt)

    l1, gi = pl.pallas_call(
        functools.partial(_box_body, inv_nb=inv_nb),
        in_specs=[
            pl.BlockSpec((B, 4, Q), lambda: (0, 0, 0)),
            pl.BlockSpec((T, 4, B), lambda: (0, 0, 0)),
            pl.BlockSpec((T, B), lambda: (0, 0)),
        ],
        out_specs=[pl.BlockSpec((1, 1), lambda: (0, 0))] * 2,
        out_shape=[jax.ShapeDtypeStruct((1, 1), f32)] * 2,
    )(pbt, tbt, sit)

    return (ce[0, 0], l1[0, 0], gi[0, 0])


# speedup vs baseline: 4.5388x; 1.1136x over previous
import functools

import jax
import jax.numpy as jnp
from jax import lax
from jax.experimental import pallas as pl
from jax.experimental.pallas import tpu as pltpu

_ALPHA = 0.25
_BB = 8


def _body(xt_ref, sit_ref, tlt_ref, pbt_ref, tbt_ref,
          ce_ref, l1_ref, gi_ref, cls_ref, *, inv_nb):
    g = pl.program_id(0)
    C, BB, Q = xt_ref.shape
    T, B = sit_ref.shape

    @pl.when(g == 0)
    def _first():
        ce_ref[...] = jnp.zeros((1, 1), jnp.float32)
        sit = sit_ref[...]
        tlt = tlt_ref[...]
        qio = lax.broadcasted_iota(jnp.int32, (T, B, Q), 2)
        tio = lax.broadcasted_iota(jnp.int32, (T, B, Q), 0)
        eq = sit[:, :, None] == qio
        t_last = jnp.max(jnp.where(eq, tio, -1), axis=0)
        pick = eq & (tio == t_last[None])
        lab = jnp.sum(jnp.where(pick, tlt[:, :, None], 0), axis=0)
        cls_ref[...] = jnp.where(t_last >= 0, lab, C)

        sel = eq.astype(jnp.float32)

        def coord(k):
            src = jnp.sum(sel * pbt_ref[:, k, :][None], axis=2)
            return src, tbt_ref[:, k, :]

        scx, tcx = coord(0)
        scy, tcy = coord(1)
        sw, tw = coord(2)
        sh, th = coord(3)
        l1 = (jnp.abs(scx - tcx) + jnp.abs(scy - tcy)
              + jnp.abs(sw - tw) + jnp.abs(sh - th))
        l1_ref[...] = (jnp.sum(l1) * inv_nb).reshape(1, 1)

        sx0, sx1 = scx - 0.5 * sw, scx + 0.5 * sw
        sy0, sy1 = scy - 0.5 * sh, scy + 0.5 * sh
        tx0, tx1 = tcx - 0.5 * tw, tcx + 0.5 * tw
        ty0, ty1 = tcy - 0.5 * th, tcy + 0.5 * th
        a1 = (sx1 - sx0) * (sy1 - sy0)
        a2 = (tx1 - tx0) * (ty1 - ty0)
        iw = jnp.maximum(jnp.minimum(sx1, tx1) - jnp.maximum(sx0, tx0), 0.0)
        ih = jnp.maximum(jnp.minimum(sy1, ty1) - jnp.maximum(sy0, ty0), 0.0)
        inter = iw * ih
        union = a1 + a2 - inter
        iou = inter / union
        ew = jnp.maximum(sx1, tx1) - jnp.minimum(sx0, tx0)
        eh = jnp.maximum(sy1, ty1) - jnp.minimum(sy0, ty0)
        ae = ew * eh
        giou = iou - (ae - union) / ae
        gi_ref[...] = (jnp.sum(1.0 - giou) * inv_nb).reshape(1, 1)

    x = xt_ref[...]
    cls_blk = cls_ref[pl.ds(pl.multiple_of(g * BB, BB), BB), :]
    cio = lax.broadcasted_iota(jnp.int32, (C, BB, Q), 0)
    mb = cio == cls_blk[None]
    e = jnp.exp(-jnp.abs(x))
    d = 1.0 + e
    l = jnp.log(d)
    mx = jnp.maximum(x, 0.0)
    ce = jnp.where(mb, mx - x, mx) + l
    r = 1.0 / d
    p = jnp.where(x >= 0.0, r, 1.0 - r)
    om = jnp.where(mb, 1.0 - p, p)
    w = jnp.where(mb, _ALPHA, 1.0 - _ALPHA)
    ce_ref[...] += (jnp.sum(w * ce * om * om) * inv_nb).reshape(1, 1)


def kernel(pred_logits, pred_boxes, tgt_boxes, src_idx, tgt_labels):
    B, Q, C = pred_logits.shape
    T = src_idx.shape[1]
    f32 = jnp.float32
    inv_nb = 1.0 / float(B * T)

    xt = jnp.transpose(pred_logits, (2, 0, 1))
    sit = jnp.transpose(src_idx, (1, 0))
    tlt = jnp.transpose(tgt_labels, (1, 0))
    pbt = jnp.transpose(pred_boxes, (0, 2, 1))
    tbt = jnp.transpose(tgt_boxes, (1, 2, 0))

    ce, l1, gi = pl.pallas_call(
        functools.partial(_body, inv_nb=inv_nb),
        grid=(B // _BB,),
        in_specs=[
            pl.BlockSpec((C, _BB, Q), lambda g: (0, g, 0)),
            pl.BlockSpec((T, B), lambda g: (0, 0)),
            pl.BlockSpec((T, B), lambda g: (0, 0)),
            pl.BlockSpec((B, 4, Q), lambda g: (0, 0, 0)),
            pl.BlockSpec((T, 4, B), lambda g: (0, 0, 0)),
        ],
        out_specs=[pl.BlockSpec((1, 1), lambda g: (0, 0))] * 3,
        out_shape=[jax.ShapeDtypeStruct((1, 1), f32)] * 3,
        scratch_shapes=[pltpu.VMEM((B, Q), jnp.int32)],
    )(xt, sit, tlt, pbt, tbt)

    return (ce[0, 0], l1[0, 0], gi[0, 0])


# speedup vs baseline: 4.8071x; 1.0591x over previous
import functools

import jax
import jax.numpy as jnp
from jax import lax
from jax.experimental import pallas as pl
from jax.experimental.pallas import tpu as pltpu

_ALPHA = 0.25
_BB = 8


def _body(xt_ref, sit_ref, tlt_ref, pbt_ref, tbt_ref,
          ce_ref, l1_ref, gi_ref, cls_ref, *, inv_nb):
    g = pl.program_id(0)
    C, BB, Q = xt_ref.shape
    T, B = sit_ref.shape

    @pl.when(g == 0)
    def _first():
        ce_ref[...] = jnp.zeros((1, 1), jnp.float32)
        sit = sit_ref[...]
        tlt = tlt_ref[...]
        qio = lax.broadcasted_iota(jnp.int32, (T, B, Q), 2)
        tio = lax.broadcasted_iota(jnp.int32, (T, B, Q), 0)
        eq = sit[:, :, None] == qio
        t_last = jnp.max(jnp.where(eq, tio, -1), axis=0)
        pick = eq & (tio == t_last[None])
        lab = jnp.sum(jnp.where(pick, tlt[:, :, None], 0), axis=0)
        cls_ref[...] = jnp.where(t_last >= 0, lab, C)

        sel = eq.astype(jnp.float32)

        def coord(k):
            src = jnp.sum(sel * pbt_ref[:, k, :][None], axis=2)
            return src, tbt_ref[:, k, :]

        scx, tcx = coord(0)
        scy, tcy = coord(1)
        sw, tw = coord(2)
        sh, th = coord(3)
        l1 = (jnp.abs(scx - tcx) + jnp.abs(scy - tcy)
              + jnp.abs(sw - tw) + jnp.abs(sh - th))
        l1_ref[...] = (jnp.sum(l1) * inv_nb).reshape(1, 1)

        sx0, sx1 = scx - 0.5 * sw, scx + 0.5 * sw
        sy0, sy1 = scy - 0.5 * sh, scy + 0.5 * sh
        tx0, tx1 = tcx - 0.5 * tw, tcx + 0.5 * tw
        ty0, ty1 = tcy - 0.5 * th, tcy + 0.5 * th
        a1 = (sx1 - sx0) * (sy1 - sy0)
        a2 = (tx1 - tx0) * (ty1 - ty0)
        iw = jnp.maximum(jnp.minimum(sx1, tx1) - jnp.maximum(sx0, tx0), 0.0)
        ih = jnp.maximum(jnp.minimum(sy1, ty1) - jnp.maximum(sy0, ty0), 0.0)
        inter = iw * ih
        union = a1 + a2 - inter
        iou = inter / union
        ew = jnp.maximum(sx1, tx1) - jnp.minimum(sx0, tx0)
        eh = jnp.maximum(sy1, ty1) - jnp.minimum(sy0, ty0)
        ae = ew * eh
        giou = iou - (ae - union) / ae
        gi_ref[...] = (jnp.sum(1.0 - giou) * inv_nb).reshape(1, 1)

    x = xt_ref[...]
    cls_blk = cls_ref[pl.ds(pl.multiple_of(g * BB, BB), BB), :]
    cio = lax.broadcasted_iota(jnp.int32, (C, BB, Q), 0)
    mb = cio == cls_blk[None]
    e = jnp.exp(-jnp.abs(x))
    d = 1.0 + e
    l = jnp.log(d)
    sp = jnp.maximum(x, 0.0) + l
    r = 1.0 / d
    p = jnp.where(x >= 0.0, r, 1.0 - r)
    acc0 = jnp.sum(p * (p * sp))
    xm = jnp.sum(jnp.where(mb, x, 0.0), axis=0)

    em = jnp.exp(-jnp.abs(xm))
    dm = 1.0 + em
    lm = jnp.log(dm)
    spm = jnp.maximum(xm, 0.0) + lm
    rm = 1.0 / dm
    pm = jnp.where(xm >= 0.0, rm, 1.0 - rm)
    omm = 1.0 - pm
    delta = (_ALPHA * omm * omm * (spm - xm)
             - (1.0 - _ALPHA) * pm * pm * spm)
    delta = jnp.where(cls_blk < C, delta, 0.0)
    ce_ref[...] += (((1.0 - _ALPHA) * acc0 + jnp.sum(delta))
                    * inv_nb).reshape(1, 1)


def kernel(pred_logits, pred_boxes, tgt_boxes, src_idx, tgt_labels):
    B, Q, C = pred_logits.shape
    T = src_idx.shape[1]
    f32 = jnp.float32
    inv_nb = 1.0 / float(B * T)

    xt = jnp.transpose(pred_logits, (2, 0, 1))
    sit = jnp.transpose(src_idx, (1, 0))
    tlt = jnp.transpose(tgt_labels, (1, 0))
    pbt = jnp.transpose(pred_boxes, (0, 2, 1))
    tbt = jnp.transpose(tgt_boxes, (1, 2, 0))

    ce, l1, gi = pl.pallas_call(
        functools.partial(_body, inv_nb=inv_nb),
        grid=(B // _BB,),
        in_specs=[
            pl.BlockSpec((C, _BB, Q), lambda g: (0, g, 0)),
            pl.BlockSpec((T, B), lambda g: (0, 0)),
            pl.BlockSpec((T, B), lambda g: (0, 0)),
            pl.BlockSpec((B, 4, Q), lambda g: (0, 0, 0)),
            pl.BlockSpec((T, 4, B), lambda g: (0, 0, 0)),
        ],
        out_specs=[pl.BlockSpec((1, 1), lambda g: (0, 0))] * 3,
        out_shape=[jax.ShapeDtypeStruct((1, 1), f32)] * 3,
        scratch_shapes=[pltpu.VMEM((B, Q), jnp.int32)],
    )(xt, sit, tlt, pbt, tbt)

    return (ce[0, 0], l1[0, 0], gi[0, 0])


# speedup vs baseline: 4.8959x; 1.0185x over previous
import functools

import jax
import jax.numpy as jnp
from jax import lax
from jax.experimental import pallas as pl
from jax.experimental.pallas import tpu as pltpu

_ALPHA = 0.25
_BB = 8


def _body(xt_ref, sit_ref, tlt_ref, pbt_ref, tbt_ref,
          ce_ref, l1_ref, gi_ref, cls_ref, *, inv_nb):
    g = pl.program_id(0)
    C, BB, Q = xt_ref.shape
    T, B = sit_ref.shape

    @pl.when(g == 0)
    def _first():
        ce_ref[...] = jnp.zeros((1, 1), jnp.float32)
        sit = sit_ref[...]
        tlt = tlt_ref[...]
        qio = lax.broadcasted_iota(jnp.int32, (T, B, Q), 2)
        eq = sit[:, :, None] == qio
        tio2 = lax.broadcasted_iota(jnp.int32, (T, B), 0)
        key = tio2 * 128 + tlt
        kmax = jnp.max(jnp.where(eq, key[:, :, None], -1), axis=0)
        cls_ref[...] = jnp.where(kmax >= 0, kmax & 127, C)

        sel = eq.astype(jnp.float32)

        def coord(k):
            src = jnp.sum(sel * pbt_ref[:, k, :][None], axis=2)
            return src, tbt_ref[:, k, :]

        scx, tcx = coord(0)
        scy, tcy = coord(1)
        sw, tw = coord(2)
        sh, th = coord(3)
        l1 = (jnp.abs(scx - tcx) + jnp.abs(scy - tcy)
              + jnp.abs(sw - tw) + jnp.abs(sh - th))
        l1_ref[...] = (jnp.sum(l1) * inv_nb).reshape(1, 1)

        sx0, sx1 = scx - 0.5 * sw, scx + 0.5 * sw
        sy0, sy1 = scy - 0.5 * sh, scy + 0.5 * sh
        tx0, tx1 = tcx - 0.5 * tw, tcx + 0.5 * tw
        ty0, ty1 = tcy - 0.5 * th, tcy + 0.5 * th
        a1 = (sx1 - sx0) * (sy1 - sy0)
        a2 = (tx1 - tx0) * (ty1 - ty0)
        iw = jnp.maximum(jnp.minimum(sx1, tx1) - jnp.maximum(sx0, tx0), 0.0)
        ih = jnp.maximum(jnp.minimum(sy1, ty1) - jnp.maximum(sy0, ty0), 0.0)
        inter = iw * ih
        union = a1 + a2 - inter
        iou = inter / union
        ew = jnp.maximum(sx1, tx1) - jnp.minimum(sx0, tx0)
        eh = jnp.maximum(sy1, ty1) - jnp.minimum(sy0, ty0)
        ae = ew * eh
        giou = iou - (ae - union) / ae
        gi_ref[...] = (jnp.sum(1.0 - giou) * inv_nb).reshape(1, 1)

    x = xt_ref[...]
    cls_blk = cls_ref[pl.ds(pl.multiple_of(g * BB, BB), BB), :]
    cio = lax.broadcasted_iota(jnp.int32, (C, BB, Q), 0)
    mb = cio == cls_blk[None]
    e = jnp.exp(-jnp.abs(x))
    d = 1.0 + e
    l = jnp.log(d)
    sp = jnp.maximum(x, 0.0) + l
    r = 1.0 / d
    p = jnp.where(x >= 0.0, r, 1.0 - r)
    acc0 = jnp.sum(p * (p * sp))
    xm = jnp.sum(jnp.where(mb, x, 0.0), axis=0)

    em = jnp.exp(-jnp.abs(xm))
    dm = 1.0 + em
    lm = jnp.log(dm)
    spm = jnp.maximum(xm, 0.0) + lm
    rm = 1.0 / dm
    pm = jnp.where(xm >= 0.0, rm, 1.0 - rm)
    omm = 1.0 - pm
    delta = (_ALPHA * omm * omm * (spm - xm)
             - (1.0 - _ALPHA) * pm * pm * spm)
    delta = jnp.where(cls_blk < C, delta, 0.0)
    ce_ref[...] += (((1.0 - _ALPHA) * acc0 + jnp.sum(delta))
                    * inv_nb).reshape(1, 1)


def kernel(pred_logits, pred_boxes, tgt_boxes, src_idx, tgt_labels):
    B, Q, C = pred_logits.shape
    T = src_idx.shape[1]
    f32 = jnp.float32
    inv_nb = 1.0 / float(B * T)

    xt = jnp.transpose(pred_logits, (2, 0, 1))
    sit = jnp.transpose(src_idx, (1, 0))
    tlt = jnp.transpose(tgt_labels, (1, 0))
    pbt = jnp.transpose(pred_boxes, (0, 2, 1))
    tbt = jnp.transpose(tgt_boxes, (1, 2, 0))

    ce, l1, gi = pl.pallas_call(
        functools.partial(_body, inv_nb=inv_nb),
        grid=(B // _BB,),
        in_specs=[
            pl.BlockSpec((C, _BB, Q), lambda g: (0, g, 0)),
            pl.BlockSpec((T, B), lambda g: (0, 0)),
            pl.BlockSpec((T, B), lambda g: (0, 0)),
            pl.BlockSpec((B, 4, Q), lambda g: (0, 0, 0)),
            pl.BlockSpec((T, 4, B), lambda g: (0, 0, 0)),
        ],
        out_specs=[pl.BlockSpec((1, 1), lambda g: (0, 0))] * 3,
        out_shape=[jax.ShapeDtypeStruct((1, 1), f32)] * 3,
        scratch_shapes=[pltpu.VMEM((B, Q), jnp.int32)],
    )(xt, sit, tlt, pbt, tbt)

    return (ce[0, 0], l1[0, 0], gi[0, 0])
